# baseline (speedup 1.0000x reference)
import functools

import jax
import jax.numpy as jnp
from jax import lax
from jax.experimental import pallas as pl
from jax.experimental.pallas import tpu as pltpu
from jax.experimental.pallas import tpu_sc as plsc

N = 10000
E = 320000
C = 128
H = 4 * C
FCO = 16 * C
OUT = 128

NC = 2
NS = 16
CH = C // NC
BLK = 128
NBLK = -(-(E // NS) // BLK)
EPT = NBLK * BLK
EPAD = NS * EPT
NBUF = 6
NMAIN = (NBLK // NBUF) * NBUF
RPT = N // NS

ROWS = 1000
GRID = N // ROWS


def _edge_agg_kernel(xcat_hbm, src0_hbm, src1_hbm, dst_hbm, zeros_hbm,
                     out_hbm, src_v, dst_v, b0, b1, b2, b3, b4, b5, acc,
                     gs0, gs1, gs2, gs3, gs4, gs5,
                     ss0, ss1, ss2, ss3, ss4, ss5):
    c = lax.axis_index("c")
    s = lax.axis_index("s")
    bufs = (b0, b1, b2, b3, b4, b5)
    gsem = (gs0, gs1, gs2, gs3, gs4, gs5)
    ssem = (ss0, ss1, ss2, ss3, ss4, ss5)
    G = NBUF // 2

    pltpu.sync_copy(zeros_hbm, acc.at[pl.ds(s * RPT, RPT)])

    @pl.when(s == 0)
    def _():
        pltpu.sync_copy(zeros_hbm.at[pl.ds(0, 8)], acc.at[pl.ds(N, 8)])

    @pl.when(c == 0)
    def _():
        pltpu.sync_copy(src0_hbm.at[s], src_v)

    @pl.when(c == 1)
    def _():
        pltpu.sync_copy(src1_hbm.at[s], src_v)

    pltpu.sync_copy(dst_hbm.at[s], dst_v)
    plsc.subcore_barrier()

    def gather(j, u):
        pltpu.async_copy(xcat_hbm.at[src_v.at[j]], bufs[u], gsem[u])

    def wait_gather(u):
        pltpu.make_async_copy(xcat_hbm.at[src_v.at[0]], bufs[u],
                              gsem[u]).wait()

    def scatter(j, u):
        pltpu.async_copy(bufs[u], acc.at[dst_v.at[j]], ssem[u], add=True)

    def wait_scatter(u):
        pltpu.make_async_copy(bufs[u], acc.at[dst_v.at[0]], ssem[u]).wait()

    def slot(j, u):
        wait_gather(u)
        scatter(j, u)
        w = (u + G) % NBUF

        @pl.when(j - G >= 0)
        def _():
            wait_scatter(w)

        @pl.when(j + G < NBLK)
        def _():
            gather(j + G, w)

    for u in range(G):
        gather(u, u)

    def body(i, carry):
        j0 = NBUF * i
        for u in range(NBUF):
            slot(j0 + u, u)
        return carry

    lax.fori_loop(0, NMAIN // NBUF, body, 0)
    for j in range(NMAIN, NBLK):
        slot(j, j % NBUF)
    for j in range(NBLK - G, NBLK):
        wait_scatter(j % NBUF)

    plsc.subcore_barrier()
    pltpu.sync_copy(acc.at[pl.ds(s * RPT, RPT)], out_hbm.at[c, s])


def _edge_agg(xcat, src0, src1, dst3, zeros):
    mesh = plsc.VectorSubcoreMesh(core_axis_name="c", subcore_axis_name="s")
    return pl.kernel(
        _edge_agg_kernel,
        out_type=jax.ShapeDtypeStruct((NC, NS, RPT, CH), jnp.float32),
        mesh=mesh,
        compiler_params=pltpu.CompilerParams(use_tc_tiling_on_sc=False),
        scratch_types=(
            [pltpu.VMEM((NBLK, BLK), jnp.int32)] * 2
            + [pltpu.VMEM((BLK, CH), jnp.float32)] * NBUF
            + [pltpu.VMEM_SHARED((N + 8, CH), jnp.float32)]
            + [pltpu.SemaphoreType.DMA] * (2 * NBUF)
        ),
    )(xcat, src0, src1, dst3, zeros)


def _stats_kernel(x_ref, agg_ref, w1a_ref, b1a_ref, w2a_ref, b2a_ref,
                  pre1_ref, pre2_ref, stats_ref):
    h = x_ref[...] + agg_ref[...]
    pre1 = jnp.dot(h, w1a_ref[...], preferred_element_type=jnp.float32)
    pre1 = pre1 + b1a_ref[...]
    pre2 = jnp.dot(h, w2a_ref[...], preferred_element_type=jnp.float32)
    pre2 = pre2 + b2a_ref[...]
    pre1_ref[...] = pre1
    pre2_ref[...] = pre2
    blk = jnp.concatenate([
        jnp.sum(pre1, axis=0, keepdims=True),
        jnp.sum(pre1 * pre1, axis=0, keepdims=True),
        jnp.sum(pre2, axis=0, keepdims=True),
        jnp.sum(pre2 * pre2, axis=0, keepdims=True),
        jnp.zeros((4, H), jnp.float32),
    ], axis=0)

    @pl.when(pl.program_id(0) == 0)
    def _():
        stats_ref[...] = jnp.zeros_like(stats_ref)

    stats_ref[...] += blk


def _stats(x, agg, w1a, b1a, w2a, b2a):
    return pl.pallas_call(
        _stats_kernel,
        grid=(GRID,),
        in_specs=[
            pl.BlockSpec((ROWS, C), lambda i: (i, 0)),
            pl.BlockSpec((ROWS, C), lambda i: (i, 0)),
            pl.BlockSpec((C, H), lambda i: (0, 0)),
            pl.BlockSpec((1, H), lambda i: (0, 0)),
            pl.BlockSpec((C, H), lambda i: (0, 0)),
            pl.BlockSpec((1, H), lambda i: (0, 0)),
        ],
        out_specs=[
            pl.BlockSpec((ROWS, H), lambda i: (i, 0)),
            pl.BlockSpec((ROWS, H), lambda i: (i, 0)),
            pl.BlockSpec((8, H), lambda i: (0, 0)),
        ],
        out_shape=[
            jax.ShapeDtypeStruct((N, H), jnp.float32),
            jax.ShapeDtypeStruct((N, H), jnp.float32),
            jax.ShapeDtypeStruct((8, H), jnp.float32),
        ],
    )(x, agg, w1a, b1a, w2a, b2a)


def _mm_kernel(a_ref, b_ref, o_ref):
    o_ref[...] = jnp.dot(a_ref[...], b_ref[...],
                         preferred_element_type=jnp.float32)


def _mm_rows(a, b, blk_rows):
    m, k = a.shape
    _, n = b.shape
    return pl.pallas_call(
        _mm_kernel,
        grid=(m // blk_rows,),
        in_specs=[
            pl.BlockSpec((blk_rows, k), lambda i: (i, 0)),
            pl.BlockSpec((k, n), lambda i: (0, 0)),
        ],
        out_specs=pl.BlockSpec((blk_rows, n), lambda i: (i, 0)),
        out_shape=jax.ShapeDtypeStruct((m, n), jnp.float32),
    )(a, b)


def _bias_kernel(l1b_ref, m_ref, l2b_ref, ow_ref, ob_ref, o_ref):
    t = jnp.dot(l1b_ref[...], m_ref[...], preferred_element_type=jnp.float32)
    t += jnp.dot(l2b_ref[...], ow_ref[...], preferred_element_type=jnp.float32)
    o_ref[...] = t + ob_ref[...]


def _compose_bias(l1_b, m, l2_b, out_w, out_b):
    return pl.pallas_call(
        _bias_kernel,
        out_shape=jax.ShapeDtypeStruct((1, OUT), jnp.float32),
    )(l1_b, m, l2_b, out_w, out_b)


def _head_kernel(pre1_ref, pre2_ref, stats_ref,
                 g1_ref, be1_ref, g2_ref, be2_ref,
                 w1b_ref, b1b_ref, w2b_ref, b2b_ref,
                 fct_ref, fcb_ref, fcbias_ref, wc_ref, bc_ref, out_ref):
    st = stats_ref[...]
    inv_n = 1.0 / N
    mu1 = st[0:1] * inv_n
    var1 = st[1:2] * inv_n - mu1 * mu1
    a1 = g1_ref[...] * lax.rsqrt(var1 + 1e-5)
    c1 = be1_ref[...] - mu1 * a1
    mu2 = st[2:3] * inv_n
    var2 = st[3:4] * inv_n - mu2 * mu2
    a2 = g2_ref[...] * lax.rsqrt(var2 + 1e-5)
    c2 = be2_ref[...] - mu2 * a2

    bf = jnp.bfloat16
    n1 = jnp.maximum(pre1_ref[...] * a1 + c1, 0.0)
    x1 = jnp.dot(n1.astype(bf), w1b_ref[...],
                 preferred_element_type=jnp.float32)
    x1 = jnp.maximum(x1 + b1b_ref[...], 0.0)
    n2 = jnp.maximum(pre2_ref[...] * a2 + c2, 0.0)
    x2 = jnp.dot(n2.astype(bf), w2b_ref[...],
                 preferred_element_type=jnp.float32)
    x2 = jnp.maximum(x2 + b2b_ref[...], 0.0)

    t = jnp.dot(x1.astype(bf), fct_ref[...],
                preferred_element_type=jnp.float32)
    t += jnp.dot(x2.astype(bf), fcb_ref[...],
                 preferred_element_type=jnp.float32)
    t += fcbias_ref[...]
    t = jnp.where(t > 0, t, 0.01 * t)

    o = jnp.dot(t.astype(bf), wc_ref[...],
                preferred_element_type=jnp.float32)
    o = o + bc_ref[...]
    out_ref[...] = 1.0 / (1.0 + jnp.exp(-o))


def _head(pre1, pre2, stats, g1, be1, g2, be2, w1b, b1b, w2b, b2b,
          fct, fcb, fc_b, wc, bc):
    vec = lambda n: pl.BlockSpec((1, n), lambda i: (0, 0))
    return pl.pallas_call(
        _head_kernel,
        grid=(GRID,),
        in_specs=[
            pl.BlockSpec((ROWS, H), lambda i: (i, 0)),
            pl.BlockSpec((ROWS, H), lambda i: (i, 0)),
            pl.BlockSpec((8, H), lambda i: (0, 0)),
            vec(H), vec(H), vec(H), vec(H),
            pl.BlockSpec((H, H), lambda i: (0, 0)), vec(H),
            pl.BlockSpec((H, H), lambda i: (0, 0)), vec(H),
            pl.BlockSpec((H, FCO), lambda i: (0, 0)),
            pl.BlockSpec((H, FCO), lambda i: (0, 0)),
            vec(FCO),
            pl.BlockSpec((FCO, OUT), lambda i: (0, 0)),
            vec(OUT),
        ],
        out_specs=pl.BlockSpec((ROWS, OUT), lambda i: (i, 0)),
        out_shape=jax.ShapeDtypeStruct((N, OUT), jnp.float32),
    )(pre1, pre2, stats, g1, be1, g2, be2, w1b, b1b, w2b, b2b,
      fct, fcb, fc_b, wc, bc)


def kernel(x, edge_index, batch,
           w1a, b1a, g1, be1, w1b, b1b,
           w2a, b2a, g2, be2, w2b, b2b,
           fc_w, fc_b, l1_w, l1_b, l2_w, l2_b, out_w, out_b):
    del batch

    npad = EPAD - E
    src_p = jnp.concatenate([edge_index[0],
                             jnp.zeros((npad,), edge_index.dtype)])
    dst_p = jnp.concatenate([edge_index[1],
                             jnp.full((npad,), N, edge_index.dtype)])
    src0 = (src_p * 2).reshape(NS, NBLK, BLK)
    src1 = src0 + 1
    dst3 = dst_p.reshape(NS, NBLK, BLK)
    xcat = x.reshape(2 * N, CH)
    zeros = jnp.zeros((RPT, CH), jnp.float32)
    halves = _edge_agg(xcat, src0, src1, dst3, zeros)
    agg = jnp.concatenate([halves[0].reshape(N, CH),
                           halves[1].reshape(N, CH)], axis=1)

    row = lambda v: v.reshape(1, -1)
    pre1, pre2, stats = _stats(x, agg, w1a, row(b1a), w2a, row(b2a))

    m = _mm_rows(l2_w, out_w, 512)
    wc = _mm_rows(l1_w, m, 512)
    bc = _compose_bias(row(l1_b), m, row(l2_b), out_w, row(out_b))

    bf = jnp.bfloat16
    return _head(pre1, pre2, stats, row(g1), row(be1), row(g2), row(be2),
                 w1b.astype(bf), row(b1b), w2b.astype(bf), row(b2b),
                 fc_w[:H].astype(bf), fc_w[H:].astype(bf), row(fc_b),
                 wc.astype(bf), bc)


# speedup vs baseline: 1.1457x; 1.1457x over previous
import functools

import jax
import jax.numpy as jnp
from jax import lax
from jax.experimental import pallas as pl
from jax.experimental.pallas import tpu as pltpu
from jax.experimental.pallas import tpu_sc as plsc

N = 10000
E = 320000
C = 128
H = 4 * C
FCO = 16 * C
OUT = 128

NC = 2
NS = 16
CH = C // NC
BLK = 80
EPT = E // NS
NBLK = EPT // BLK
NBUF = 6
NMAIN = (NBLK // NBUF) * NBUF
RPT = N // NS

ROWS = 2000
GRID = N // ROWS


def _edge_agg_kernel(xcat_hbm, src0_hbm, src1_hbm, dst_hbm, zeros_hbm,
                     out_hbm, src_v, dst_v, b0, b1, b2, b3, b4, b5, acc,
                     gs0, gs1, gs2, gs3, gs4, gs5,
                     ss0, ss1, ss2, ss3, ss4, ss5):
    c = lax.axis_index("c")
    s = lax.axis_index("s")
    bufs = (b0, b1, b2, b3, b4, b5)
    gsem = (gs0, gs1, gs2, gs3, gs4, gs5)
    ssem = (ss0, ss1, ss2, ss3, ss4, ss5)
    G = NBUF // 2

    pltpu.sync_copy(zeros_hbm, acc.at[pl.ds(s * RPT, RPT)])

    @pl.when(s == 0)
    def _():
        pltpu.sync_copy(zeros_hbm.at[pl.ds(0, 8)], acc.at[pl.ds(N, 8)])

    @pl.when(c == 0)
    def _():
        pltpu.sync_copy(src0_hbm.at[s], src_v)

    @pl.when(c == 1)
    def _():
        pltpu.sync_copy(src1_hbm.at[s], src_v)

    pltpu.sync_copy(dst_hbm.at[s], dst_v)
    plsc.subcore_barrier()

    def gather(j, u):
        pltpu.async_copy(xcat_hbm.at[src_v.at[j]], bufs[u], gsem[u])

    def wait_gather(u):
        pltpu.make_async_copy(xcat_hbm.at[src_v.at[0]], bufs[u],
                              gsem[u]).wait()

    def scatter(j, u):
        pltpu.async_copy(bufs[u], acc.at[dst_v.at[j]], ssem[u], add=True)

    def wait_scatter(u):
        pltpu.make_async_copy(bufs[u], acc.at[dst_v.at[0]], ssem[u]).wait()

    def slot(j, u):
        wait_gather(u)
        scatter(j, u)
        w = (u + G) % NBUF

        @pl.when(j - G >= 0)
        def _():
            wait_scatter(w)

        @pl.when(j + G < NBLK)
        def _():
            gather(j + G, w)

    for u in range(G):
        gather(u, u)

    def body(i, carry):
        j0 = NBUF * i
        for u in range(NBUF):
            slot(j0 + u, u)
        return carry

    lax.fori_loop(0, NMAIN // NBUF, body, 0)
    for j in range(NMAIN, NBLK):
        slot(j, j % NBUF)
    for j in range(NBLK - G, NBLK):
        wait_scatter(j % NBUF)

    plsc.subcore_barrier()
    pltpu.sync_copy(acc.at[pl.ds(s * RPT, RPT)], out_hbm.at[c, s])


def _edge_agg(xcat, src0, src1, dst3, zeros):
    mesh = plsc.VectorSubcoreMesh(core_axis_name="c", subcore_axis_name="s")
    return pl.kernel(
        _edge_agg_kernel,
        out_type=jax.ShapeDtypeStruct((NC, NS, RPT, CH), jnp.float32),
        mesh=mesh,
        compiler_params=pltpu.CompilerParams(use_tc_tiling_on_sc=False),
        scratch_types=(
            [pltpu.VMEM((NBLK, BLK), jnp.int32)] * 2
            + [pltpu.VMEM((BLK, CH), jnp.float32)] * NBUF
            + [pltpu.VMEM_SHARED((N + 8, CH), jnp.float32)]
            + [pltpu.SemaphoreType.DMA] * (2 * NBUF)
        ),
    )(xcat, src0, src1, dst3, zeros)


def _stats_kernel(x_ref, agg_ref, w1a_ref, b1a_ref, w2a_ref, b2a_ref,
                  pre1_ref, pre2_ref, stats_ref):
    h = x_ref[...] + agg_ref[...]
    pre1 = jnp.dot(h, w1a_ref[...], preferred_element_type=jnp.float32)
    pre1 = pre1 + b1a_ref[...]
    pre2 = jnp.dot(h, w2a_ref[...], preferred_element_type=jnp.float32)
    pre2 = pre2 + b2a_ref[...]
    pre1_ref[...] = pre1
    pre2_ref[...] = pre2
    blk = jnp.concatenate([
        jnp.sum(pre1, axis=0, keepdims=True),
        jnp.sum(pre1 * pre1, axis=0, keepdims=True),
        jnp.sum(pre2, axis=0, keepdims=True),
        jnp.sum(pre2 * pre2, axis=0, keepdims=True),
        jnp.zeros((4, H), jnp.float32),
    ], axis=0)

    @pl.when(pl.program_id(0) == 0)
    def _():
        stats_ref[...] = jnp.zeros_like(stats_ref)

    stats_ref[...] += blk


def _stats(x, agg, w1a, b1a, w2a, b2a):
    return pl.pallas_call(
        _stats_kernel,
        grid=(GRID,),
        in_specs=[
            pl.BlockSpec((ROWS, C), lambda i: (i, 0)),
            pl.BlockSpec((ROWS, C), lambda i: (i, 0)),
            pl.BlockSpec((C, H), lambda i: (0, 0)),
            pl.BlockSpec((1, H), lambda i: (0, 0)),
            pl.BlockSpec((C, H), lambda i: (0, 0)),
            pl.BlockSpec((1, H), lambda i: (0, 0)),
        ],
        out_specs=[
            pl.BlockSpec((ROWS, H), lambda i: (i, 0)),
            pl.BlockSpec((ROWS, H), lambda i: (i, 0)),
            pl.BlockSpec((8, H), lambda i: (0, 0)),
        ],
        out_shape=[
            jax.ShapeDtypeStruct((N, H), jnp.float32),
            jax.ShapeDtypeStruct((N, H), jnp.float32),
            jax.ShapeDtypeStruct((8, H), jnp.float32),
        ],
    )(x, agg, w1a, b1a, w2a, b2a)


def _mm_kernel(a_ref, b_ref, o_ref):
    o_ref[...] = jnp.dot(a_ref[...], b_ref[...],
                         preferred_element_type=jnp.float32)


def _mm_rows(a, b, blk_rows):
    m, k = a.shape
    _, n = b.shape
    return pl.pallas_call(
        _mm_kernel,
        grid=(m // blk_rows,),
        in_specs=[
            pl.BlockSpec((blk_rows, k), lambda i: (i, 0)),
            pl.BlockSpec((k, n), lambda i: (0, 0)),
        ],
        out_specs=pl.BlockSpec((blk_rows, n), lambda i: (i, 0)),
        out_shape=jax.ShapeDtypeStruct((m, n), jnp.float32),
    )(a, b)


def _bias_kernel(l1b_ref, m_ref, l2b_ref, ow_ref, ob_ref, o_ref):
    t = jnp.dot(l1b_ref[...], m_ref[...], preferred_element_type=jnp.float32)
    t += jnp.dot(l2b_ref[...], ow_ref[...], preferred_element_type=jnp.float32)
    o_ref[...] = t + ob_ref[...]


def _compose_bias(l1_b, m, l2_b, out_w, out_b):
    return pl.pallas_call(
        _bias_kernel,
        out_shape=jax.ShapeDtypeStruct((1, OUT), jnp.float32),
    )(l1_b, m, l2_b, out_w, out_b)


def _head_kernel(pre1_ref, pre2_ref, stats_ref,
                 g1_ref, be1_ref, g2_ref, be2_ref,
                 w1b_ref, b1b_ref, w2b_ref, b2b_ref,
                 fct_ref, fcb_ref, fcbias_ref, wc_ref, bc_ref, out_ref):
    st = stats_ref[...]
    inv_n = 1.0 / N
    mu1 = st[0:1] * inv_n
    var1 = st[1:2] * inv_n - mu1 * mu1
    a1 = g1_ref[...] * lax.rsqrt(var1 + 1e-5)
    c1 = be1_ref[...] - mu1 * a1
    mu2 = st[2:3] * inv_n
    var2 = st[3:4] * inv_n - mu2 * mu2
    a2 = g2_ref[...] * lax.rsqrt(var2 + 1e-5)
    c2 = be2_ref[...] - mu2 * a2

    bf = jnp.bfloat16
    n1 = jnp.maximum(pre1_ref[...] * a1 + c1, 0.0)
    x1 = jnp.dot(n1.astype(bf), w1b_ref[...],
                 preferred_element_type=jnp.float32)
    x1 = jnp.maximum(x1 + b1b_ref[...], 0.0)
    n2 = jnp.maximum(pre2_ref[...] * a2 + c2, 0.0)
    x2 = jnp.dot(n2.astype(bf), w2b_ref[...],
                 preferred_element_type=jnp.float32)
    x2 = jnp.maximum(x2 + b2b_ref[...], 0.0)

    t = jnp.dot(x1.astype(bf), fct_ref[...],
                preferred_element_type=jnp.float32)
    t += jnp.dot(x2.astype(bf), fcb_ref[...],
                 preferred_element_type=jnp.float32)
    t += fcbias_ref[...]
    t = jnp.where(t > 0, t, 0.01 * t)

    o = jnp.dot(t.astype(bf), wc_ref[...],
                preferred_element_type=jnp.float32)
    o = o + bc_ref[...]
    out_ref[...] = 1.0 / (1.0 + jnp.exp(-o))


def _head(pre1, pre2, stats, g1, be1, g2, be2, w1b, b1b, w2b, b2b,
          fct, fcb, fc_b, wc, bc):
    vec = lambda n: pl.BlockSpec((1, n), lambda i: (0, 0))
    return pl.pallas_call(
        _head_kernel,
        grid=(GRID,),
        in_specs=[
            pl.BlockSpec((ROWS, H), lambda i: (i, 0)),
            pl.BlockSpec((ROWS, H), lambda i: (i, 0)),
            pl.BlockSpec((8, H), lambda i: (0, 0)),
            vec(H), vec(H), vec(H), vec(H),
            pl.BlockSpec((H, H), lambda i: (0, 0)), vec(H),
            pl.BlockSpec((H, H), lambda i: (0, 0)), vec(H),
            pl.BlockSpec((H, FCO), lambda i: (0, 0)),
            pl.BlockSpec((H, FCO), lambda i: (0, 0)),
            vec(FCO),
            pl.BlockSpec((FCO, OUT), lambda i: (0, 0)),
            vec(OUT),
        ],
        out_specs=pl.BlockSpec((ROWS, OUT), lambda i: (i, 0)),
        out_shape=jax.ShapeDtypeStruct((N, OUT), jnp.float32),
    )(pre1, pre2, stats, g1, be1, g2, be2, w1b, b1b, w2b, b2b,
      fct, fcb, fc_b, wc, bc)


def kernel(x, edge_index, batch,
           w1a, b1a, g1, be1, w1b, b1b,
           w2a, b2a, g2, be2, w2b, b2b,
           fc_w, fc_b, l1_w, l1_b, l2_w, l2_b, out_w, out_b):
    del batch

    src0 = (edge_index[0] * 2).reshape(NS, NBLK, BLK)
    src1 = src0 + 1
    dst3 = edge_index[1].reshape(NS, NBLK, BLK)
    xcat = x.reshape(2 * N, CH)
    zeros = jnp.zeros((RPT, CH), jnp.float32)
    halves = _edge_agg(xcat, src0, src1, dst3, zeros)
    agg = jnp.concatenate([halves[0].reshape(N, CH),
                           halves[1].reshape(N, CH)], axis=1)

    row = lambda v: v.reshape(1, -1)
    pre1, pre2, stats = _stats(x, agg, w1a, row(b1a), w2a, row(b2a))

    m = _mm_rows(l2_w, out_w, 512)
    wc = _mm_rows(l1_w, m, 512)
    bc = _compose_bias(row(l1_b), m, row(l2_b), out_w, row(out_b))

    bf = jnp.bfloat16
    return _head(pre1, pre2, stats, row(g1), row(be1), row(g2), row(be2),
                 w1b.astype(bf), row(b1b), w2b.astype(bf), row(b2b),
                 fc_w[:H].astype(bf), fc_w[H:].astype(bf), row(fc_b),
                 wc.astype(bf), bc)


# speedup vs baseline: 1.1768x; 1.0272x over previous
import functools

import jax
import jax.numpy as jnp
from jax import lax
from jax.experimental import pallas as pl
from jax.experimental.pallas import tpu as pltpu
from jax.experimental.pallas import tpu_sc as plsc

N = 10000
E = 320000
C = 128
H = 4 * C
FCO = 16 * C
OUT = 128

NC = 2
NS = 16
CH = C // NC
BLK = 80
EPT = E // NS
NBLK = EPT // BLK
NBUF = 6
NMAIN = (NBLK // NBUF) * NBUF
RPT = N // NS

ROWS = 2000
GRID = N // ROWS


def _edge_agg_kernel(xcat_hbm, src0_hbm, src1_hbm, dst_hbm, zeros_hbm,
                     out_hbm, src_v, dst_v, b0, b1, b2, b3, b4, b5, acc,
                     gs0, gs1, gs2, gs3, gs4, gs5,
                     ss0, ss1, ss2, ss3, ss4, ss5):
    c = lax.axis_index("c")
    s = lax.axis_index("s")
    bufs = (b0, b1, b2, b3, b4, b5)
    gsem = (gs0, gs1, gs2, gs3, gs4, gs5)
    ssem = (ss0, ss1, ss2, ss3, ss4, ss5)
    G = NBUF // 2

    pltpu.sync_copy(zeros_hbm, acc.at[pl.ds(s * RPT, RPT)])

    @pl.when(s == 0)
    def _():
        pltpu.sync_copy(zeros_hbm.at[pl.ds(0, 8)], acc.at[pl.ds(N, 8)])

    @pl.when(c == 0)
    def _():
        pltpu.sync_copy(src0_hbm.at[s], src_v)

    @pl.when(c == 1)
    def _():
        pltpu.sync_copy(src1_hbm.at[s], src_v)

    pltpu.sync_copy(dst_hbm.at[s], dst_v)
    plsc.subcore_barrier()

    def gather(j, u):
        pltpu.async_copy(xcat_hbm.at[src_v.at[j]], bufs[u], gsem[u])

    def wait_gather(u):
        pltpu.make_async_copy(xcat_hbm.at[src_v.at[0]], bufs[u],
                              gsem[u]).wait()

    def scatter(j, u):
        pltpu.async_copy(bufs[u], acc.at[dst_v.at[j]], ssem[u], add=True)

    def wait_scatter(u):
        pltpu.make_async_copy(bufs[u], acc.at[dst_v.at[0]], ssem[u]).wait()

    def slot(j, u):
        wait_gather(u)
        scatter(j, u)
        w = (u + G) % NBUF

        @pl.when(j - G >= 0)
        def _():
            wait_scatter(w)

        @pl.when(j + G < NBLK)
        def _():
            gather(j + G, w)

    for u in range(G):
        gather(u, u)

    def body(i, carry):
        j0 = NBUF * i
        for u in range(NBUF):
            slot(j0 + u, u)
        return carry

    lax.fori_loop(0, NMAIN // NBUF, body, 0)
    for j in range(NMAIN, NBLK):
        slot(j, j % NBUF)
    for j in range(NBLK - G, NBLK):
        wait_scatter(j % NBUF)

    plsc.subcore_barrier()
    pltpu.sync_copy(acc.at[pl.ds(s * RPT, RPT)], out_hbm.at[c, s])


def _edge_agg(xcat, src0, src1, dst3, zeros):
    mesh = plsc.VectorSubcoreMesh(core_axis_name="c", subcore_axis_name="s")
    return pl.kernel(
        _edge_agg_kernel,
        out_type=jax.ShapeDtypeStruct((NC, NS, RPT, CH), jnp.float32),
        mesh=mesh,
        compiler_params=pltpu.CompilerParams(use_tc_tiling_on_sc=False),
        scratch_types=(
            [pltpu.VMEM((NBLK, BLK), jnp.int32)] * 2
            + [pltpu.VMEM((BLK, CH), jnp.float32)] * NBUF
            + [pltpu.VMEM_SHARED((N + 8, CH), jnp.float32)]
            + [pltpu.SemaphoreType.DMA] * (2 * NBUF)
        ),
    )(xcat, src0, src1, dst3, zeros)


def _pre_dots(h_bf, w1a_ref, b1a_ref, w2a_ref, b2a_ref):
    bf = jnp.bfloat16
    pre1 = jnp.dot(h_bf, w1a_ref[...].astype(bf),
                   preferred_element_type=jnp.float32) + b1a_ref[...]
    pre2 = jnp.dot(h_bf, w2a_ref[...].astype(bf),
                   preferred_element_type=jnp.float32) + b2a_ref[...]
    return pre1, pre2


def _stats_kernel(x_ref, agg_ref, w1a_ref, b1a_ref, w2a_ref, b2a_ref,
                  h_ref, stats_ref):
    h = x_ref[...] + agg_ref[...]
    h_ref[...] = h
    pre1, pre2 = _pre_dots(h.astype(jnp.bfloat16),
                           w1a_ref, b1a_ref, w2a_ref, b2a_ref)
    blk = jnp.concatenate([
        jnp.sum(pre1, axis=0, keepdims=True),
        jnp.sum(pre1 * pre1, axis=0, keepdims=True),
        jnp.sum(pre2, axis=0, keepdims=True),
        jnp.sum(pre2 * pre2, axis=0, keepdims=True),
        jnp.zeros((4, H), jnp.float32),
    ], axis=0)

    @pl.when(pl.program_id(0) == 0)
    def _():
        stats_ref[...] = jnp.zeros_like(stats_ref)

    stats_ref[...] += blk


def _stats(x, agg, w1a, b1a, w2a, b2a):
    return pl.pallas_call(
        _stats_kernel,
        grid=(GRID,),
        in_specs=[
            pl.BlockSpec((ROWS, C), lambda i: (i, 0)),
            pl.BlockSpec((ROWS, C), lambda i: (i, 0)),
            pl.BlockSpec((C, H), lambda i: (0, 0)),
            pl.BlockSpec((1, H), lambda i: (0, 0)),
            pl.BlockSpec((C, H), lambda i: (0, 0)),
            pl.BlockSpec((1, H), lambda i: (0, 0)),
        ],
        out_specs=[
            pl.BlockSpec((ROWS, C), lambda i: (i, 0)),
            pl.BlockSpec((8, H), lambda i: (0, 0)),
        ],
        out_shape=[
            jax.ShapeDtypeStruct((N, C), jnp.float32),
            jax.ShapeDtypeStruct((8, H), jnp.float32),
        ],
    )(x, agg, w1a, b1a, w2a, b2a)


def _mm_kernel(a_ref, b_ref, o_ref):
    o_ref[...] = jnp.dot(a_ref[...], b_ref[...],
                         preferred_element_type=jnp.float32)


def _mm_rows(a, b, blk_rows):
    m, k = a.shape
    _, n = b.shape
    return pl.pallas_call(
        _mm_kernel,
        grid=(m // blk_rows,),
        in_specs=[
            pl.BlockSpec((blk_rows, k), lambda i: (i, 0)),
            pl.BlockSpec((k, n), lambda i: (0, 0)),
        ],
        out_specs=pl.BlockSpec((blk_rows, n), lambda i: (i, 0)),
        out_shape=jax.ShapeDtypeStruct((m, n), jnp.float32),
    )(a, b)


def _wc_kernel(l1w_ref, m_ref, l1b_ref, l2b_ref, ow_ref, ob_ref,
               wc_ref, bc_ref):
    wc_ref[...] = jnp.dot(l1w_ref[...], m_ref[...],
                          preferred_element_type=jnp.float32)

    @pl.when(pl.program_id(0) == 0)
    def _():
        t = jnp.dot(l1b_ref[...], m_ref[...],
                    preferred_element_type=jnp.float32)
        t += jnp.dot(l2b_ref[...], ow_ref[...],
                     preferred_element_type=jnp.float32)
        bc_ref[...] = t + ob_ref[...]


def _compose_wc(l1_w, m, l1_b, l2_b, out_w, out_b, blk_rows=512):
    k = l1_w.shape[1]
    return pl.pallas_call(
        _wc_kernel,
        grid=(l1_w.shape[0] // blk_rows,),
        in_specs=[
            pl.BlockSpec((blk_rows, k), lambda i: (i, 0)),
            pl.BlockSpec((k, OUT), lambda i: (0, 0)),
            pl.BlockSpec((1, k), lambda i: (0, 0)),
            pl.BlockSpec((1, FCO), lambda i: (0, 0)),
            pl.BlockSpec((FCO, OUT), lambda i: (0, 0)),
            pl.BlockSpec((1, OUT), lambda i: (0, 0)),
        ],
        out_specs=[
            pl.BlockSpec((blk_rows, OUT), lambda i: (i, 0)),
            pl.BlockSpec((1, OUT), lambda i: (0, 0)),
        ],
        out_shape=[
            jax.ShapeDtypeStruct((l1_w.shape[0], OUT), jnp.float32),
            jax.ShapeDtypeStruct((1, OUT), jnp.float32),
        ],
    )(l1_w, m, l1_b, l2_b, out_w, out_b)


def _head_kernel(h_ref, stats_ref, w1a_ref, b1a_ref, w2a_ref, b2a_ref,
                 g1_ref, be1_ref, g2_ref, be2_ref,
                 w1b_ref, b1b_ref, w2b_ref, b2b_ref,
                 fcw_ref, fcbias_ref, wc_ref, bc_ref, out_ref):
    st = stats_ref[...]
    inv_n = 1.0 / N
    mu1 = st[0:1] * inv_n
    var1 = st[1:2] * inv_n - mu1 * mu1
    a1 = g1_ref[...] * lax.rsqrt(var1 + 1e-5)
    c1 = be1_ref[...] - mu1 * a1
    mu2 = st[2:3] * inv_n
    var2 = st[3:4] * inv_n - mu2 * mu2
    a2 = g2_ref[...] * lax.rsqrt(var2 + 1e-5)
    c2 = be2_ref[...] - mu2 * a2

    bf = jnp.bfloat16
    pre1, pre2 = _pre_dots(h_ref[...].astype(bf),
                           w1a_ref, b1a_ref, w2a_ref, b2a_ref)
    n1 = jnp.maximum(pre1 * a1 + c1, 0.0)
    x1 = jnp.dot(n1.astype(bf), w1b_ref[...].astype(bf),
                 preferred_element_type=jnp.float32)
    x1 = jnp.maximum(x1 + b1b_ref[...], 0.0)
    n2 = jnp.maximum(pre2 * a2 + c2, 0.0)
    x2 = jnp.dot(n2.astype(bf), w2b_ref[...].astype(bf),
                 preferred_element_type=jnp.float32)
    x2 = jnp.maximum(x2 + b2b_ref[...], 0.0)

    xc = jnp.concatenate([x1, x2], axis=1).astype(bf)
    t = jnp.dot(xc, fcw_ref[...].astype(bf),
                preferred_element_type=jnp.float32)
    t += fcbias_ref[...]
    t = jnp.where(t > 0, t, 0.01 * t)

    o = jnp.dot(t.astype(bf), wc_ref[...].astype(bf),
                preferred_element_type=jnp.float32)
    o = o + bc_ref[...]
    out_ref[...] = 1.0 / (1.0 + jnp.exp(-o))


def _head(h, stats, w1a, b1a, w2a, b2a, g1, be1, g2, be2,
          w1b, b1b, w2b, b2b, fc_w, fc_b, wc, bc):
    vec = lambda n: pl.BlockSpec((1, n), lambda i: (0, 0))
    return pl.pallas_call(
        _head_kernel,
        grid=(GRID,),
        in_specs=[
            pl.BlockSpec((ROWS, C), lambda i: (i, 0)),
            pl.BlockSpec((8, H), lambda i: (0, 0)),
            pl.BlockSpec((C, H), lambda i: (0, 0)), vec(H),
            pl.BlockSpec((C, H), lambda i: (0, 0)), vec(H),
            vec(H), vec(H), vec(H), vec(H),
            pl.BlockSpec((H, H), lambda i: (0, 0)), vec(H),
            pl.BlockSpec((H, H), lambda i: (0, 0)), vec(H),
            pl.BlockSpec((2 * H, FCO), lambda i: (0, 0)),
            vec(FCO),
            pl.BlockSpec((FCO, OUT), lambda i: (0, 0)),
            vec(OUT),
        ],
        out_specs=pl.BlockSpec((ROWS, OUT), lambda i: (i, 0)),
        out_shape=jax.ShapeDtypeStruct((N, OUT), jnp.float32),
    )(h, stats, w1a, b1a, w2a, b2a, g1, be1, g2, be2,
      w1b, b1b, w2b, b2b, fc_w, fc_b, wc, bc)


def kernel(x, edge_index, batch,
           w1a, b1a, g1, be1, w1b, b1b,
           w2a, b2a, g2, be2, w2b, b2b,
           fc_w, fc_b, l1_w, l1_b, l2_w, l2_b, out_w, out_b):
    del batch

    src0 = (edge_index[0] * 2).reshape(NS, NBLK, BLK)
    src1 = src0 + 1
    dst3 = edge_index[1].reshape(NS, NBLK, BLK)
    xcat = x.reshape(2 * N, CH)
    zeros = jnp.zeros((RPT, CH), jnp.float32)
    halves = _edge_agg(xcat, src0, src1, dst3, zeros)
    agg = jnp.concatenate([halves[0].reshape(N, CH),
                           halves[1].reshape(N, CH)], axis=1)

    row = lambda v: v.reshape(1, -1)
    h, stats = _stats(x, agg, w1a, row(b1a), w2a, row(b2a))

    m = _mm_rows(l2_w, out_w, 512)
    wc, bc = _compose_wc(l1_w, m, row(l1_b), row(l2_b), out_w, row(out_b))

    return _head(h, stats, w1a, row(b1a), w2a, row(b2a),
                 row(g1), row(be1), row(g2), row(be2),
                 w1b, row(b1b), w2b, row(b2b),
                 fc_w, row(fc_b), wc, bc)
